# baseline (speedup 1.0000x reference)
import functools

import jax
import jax.numpy as jnp
from jax import lax
from jax.experimental import pallas as pl
from jax.experimental.pallas import tpu as pltpu
from jax.experimental.pallas import tpu_sc as plsc

D_MODEL = 768
LANES = 16
NUM_CORES = 2
NUM_SUBCORES = 16
NUM_WORKERS = NUM_CORES * NUM_SUBCORES
CHUNK_ROWS = 16


def _sc_body(
    x_hbm, pos_hbm, scale_hbm, out_hbm,
    pbuf0, pbuf1, xin0, xin1, xout0, xout1, sbuf,
    insem0, insem1, outsem0, outsem1, possem0, possem1,
):
    wid = lax.axis_index("s") * NUM_CORES + lax.axis_index("c")
    B, S, _ = x_hbm.shape
    pos_rows_per_worker = S // NUM_WORKERS
    prow0 = wid * pos_rows_per_worker
    num_pc = pos_rows_per_worker // CHUNK_ROWS
    num_pairs = num_pc * B

    pbufs = (pbuf0, pbuf1)
    xins = (xin0, xin1)
    xouts = (xout0, xout1)
    insems = (insem0, insem1)
    outsems = (outsem0, outsem1)
    possems = (possem0, possem1)

    def start_in(t, j):
        b = t % B
        row = prow0 + (t // B) * CHUNK_ROWS
        pltpu.make_async_copy(
            x_hbm.at[b, pl.ds(row, CHUNK_ROWS), :], xins[j], insems[j]
        ).start()

    def start_pos(pc, q):
        pltpu.make_async_copy(
            pos_hbm.at[pl.ds(prow0 + pc * CHUNK_ROWS, CHUNK_ROWS), :],
            pbufs[q],
            possems[q],
        ).start()

    def wait_pos(q):
        pltpu.make_async_copy(
            pos_hbm.at[pl.ds(0, CHUNK_ROWS), :], pbufs[q], possems[q]
        ).wait()

    pltpu.sync_copy(scale_hbm, sbuf)
    sv = sbuf[...]

    start_pos(0, 0)
    start_in(0, 0)
    start_in(1, 1)

    def group_body(g, carry):
        t0 = 8 * g
        for u in range(8):
            t = t0 + u
            j = u % 2
            q = u // B

            if u == 0:
                wait_pos(0)
                start_pos(2 * g + 1, 1)
            elif u == B:
                wait_pos(1)

                @pl.when(2 * g + 2 < num_pc)
                def _():
                    start_pos(2 * g + 2, 0)

            pltpu.make_async_copy(
                x_hbm.at[0, pl.ds(0, CHUNK_ROWS), :], xins[j], insems[j]
            ).wait()

            if u < 2:
                @pl.when(t >= 2)
                def _():
                    pltpu.make_async_copy(
                        xouts[j], out_hbm.at[0, pl.ds(0, CHUNK_ROWS), :],
                        outsems[j],
                    ).wait()
            else:
                pltpu.make_async_copy(
                    xouts[j], out_hbm.at[0, pl.ds(0, CHUNK_ROWS), :], outsems[j]
                ).wait()

            xin = xins[j]
            xout = xouts[j]
            pbuf = pbufs[q]

            @plsc.parallel_loop(0, CHUNK_ROWS, 1, unroll=2)
            def _(r):
                for v in range(D_MODEL // LANES):
                    sl = pl.ds(v * LANES, LANES)
                    xout[r, sl] = xin[r, sl] + pbuf[r, sl] * sv

            b = t % B
            row = prow0 + (t // B) * CHUNK_ROWS
            pltpu.make_async_copy(
                xout, out_hbm.at[b, pl.ds(row, CHUNK_ROWS), :], outsems[j]
            ).start()

            @pl.when(t + 2 < num_pairs)
            def _():
                start_in(t + 2, j)
        return carry

    lax.fori_loop(0, num_pairs // 8, group_body, 0)

    for j in range(2):
        pltpu.make_async_copy(
            xouts[j], out_hbm.at[0, pl.ds(0, CHUNK_ROWS), :], outsems[j]
        ).wait()


def kernel(x, pos_embed, scale):
    B, S, D = x.shape
    mesh = plsc.VectorSubcoreMesh(core_axis_name="c", subcore_axis_name="s")

    sc_call = functools.partial(
        pl.kernel,
        mesh=mesh,
        out_type=jax.ShapeDtypeStruct((B, S, D), jnp.float32),
        compiler_params=pltpu.CompilerParams(use_tc_tiling_on_sc=True),
        scratch_types=[
            pltpu.VMEM((CHUNK_ROWS, D_MODEL), jnp.float32),
            pltpu.VMEM((CHUNK_ROWS, D_MODEL), jnp.float32),
            pltpu.VMEM((CHUNK_ROWS, D_MODEL), jnp.float32),
            pltpu.VMEM((CHUNK_ROWS, D_MODEL), jnp.float32),
            pltpu.VMEM((CHUNK_ROWS, D_MODEL), jnp.float32),
            pltpu.VMEM((CHUNK_ROWS, D_MODEL), jnp.float32),
            pltpu.VMEM((LANES,), jnp.float32),
            pltpu.SemaphoreType.DMA,
            pltpu.SemaphoreType.DMA,
            pltpu.SemaphoreType.DMA,
            pltpu.SemaphoreType.DMA,
            pltpu.SemaphoreType.DMA,
            pltpu.SemaphoreType.DMA,
        ],
    )(_sc_body)

    return sc_call(x, pos_embed[:S], jnp.broadcast_to(scale, (LANES,)))


# speedup vs baseline: 1.0452x; 1.0452x over previous
import functools

import jax
import jax.numpy as jnp
from jax import lax
from jax.experimental import pallas as pl
from jax.experimental.pallas import tpu as pltpu
from jax.experimental.pallas import tpu_sc as plsc

D_MODEL = 768
LANES = 16
NUM_CORES = 2
NUM_SUBCORES = 16
NUM_WORKERS = NUM_CORES * NUM_SUBCORES
CHUNK_ROWS = 32


def _sc_body(
    x_hbm, pos_hbm, scale_hbm, out_hbm,
    pbuf, xin0, xin1, xout0, xout1, sbuf,
    insem0, insem1, outsem0, outsem1,
):
    wid = lax.axis_index("s") * NUM_CORES + lax.axis_index("c")
    B, S, _ = x_hbm.shape
    pos_rows_per_worker = S // NUM_WORKERS
    prow0 = wid * pos_rows_per_worker
    num_pc = pos_rows_per_worker // CHUNK_ROWS
    num_pairs = num_pc * B

    xins = (xin0, xin1)
    xouts = (xout0, xout1)
    insems = (insem0, insem1)
    outsems = (outsem0, outsem1)

    def start_in(t, j):
        b = t % B
        row = prow0 + (t // B) * CHUNK_ROWS
        pltpu.make_async_copy(
            x_hbm.at[b, pl.ds(row, CHUNK_ROWS), :], xins[j], insems[j]
        ).start()

    pltpu.sync_copy(scale_hbm, sbuf)
    sv = sbuf[...]

    start_in(0, 0)
    start_in(1, 1)

    def pair_body(g, carry):
        for j in range(2):
            t = 2 * g + j

            if j == 0:
                @pl.when(t % B == 0)
                def _():
                    pltpu.sync_copy(
                        pos_hbm.at[
                            pl.ds(prow0 + (t // B) * CHUNK_ROWS, CHUNK_ROWS), :
                        ],
                        pbuf,
                    )

            pltpu.make_async_copy(
                x_hbm.at[0, pl.ds(0, CHUNK_ROWS), :], xins[j], insems[j]
            ).wait()

            @pl.when(t >= 2)
            def _():
                pltpu.make_async_copy(
                    xouts[j], out_hbm.at[0, pl.ds(0, CHUNK_ROWS), :], outsems[j]
                ).wait()

            xin = xins[j]
            xout = xouts[j]

            @plsc.parallel_loop(0, CHUNK_ROWS, 1, unroll=2)
            def _(r):
                for u in range(D_MODEL // LANES):
                    sl = pl.ds(u * LANES, LANES)
                    xout[r, sl] = xin[r, sl] + pbuf[r, sl] * sv

            b = t % B
            row = prow0 + (t // B) * CHUNK_ROWS
            pltpu.make_async_copy(
                xout, out_hbm.at[b, pl.ds(row, CHUNK_ROWS), :], outsems[j]
            ).start()

            @pl.when(t + 2 < num_pairs)
            def _():
                start_in(t + 2, j)
        return carry

    lax.fori_loop(0, num_pairs // 2, pair_body, 0)

    for j in range(2):
        pltpu.make_async_copy(
            xouts[j], out_hbm.at[0, pl.ds(0, CHUNK_ROWS), :], outsems[j]
        ).wait()


def kernel(x, pos_embed, scale):
    B, S, D = x.shape
    mesh = plsc.VectorSubcoreMesh(core_axis_name="c", subcore_axis_name="s")

    sc_call = functools.partial(
        pl.kernel,
        mesh=mesh,
        out_type=jax.ShapeDtypeStruct((B, S, D), jnp.float32),
        compiler_params=pltpu.CompilerParams(use_tc_tiling_on_sc=True),
        scratch_types=[
            pltpu.VMEM((CHUNK_ROWS, D_MODEL), jnp.float32),
            pltpu.VMEM((CHUNK_ROWS, D_MODEL), jnp.float32),
            pltpu.VMEM((CHUNK_ROWS, D_MODEL), jnp.float32),
            pltpu.VMEM((CHUNK_ROWS, D_MODEL), jnp.float32),
            pltpu.VMEM((CHUNK_ROWS, D_MODEL), jnp.float32),
            pltpu.VMEM((LANES,), jnp.float32),
            pltpu.SemaphoreType.DMA,
            pltpu.SemaphoreType.DMA,
            pltpu.SemaphoreType.DMA,
            pltpu.SemaphoreType.DMA,
        ],
    )(_sc_body)

    return sc_call(x, pos_embed[:S], jnp.broadcast_to(scale, (LANES,)))
